# Initial kernel scaffold; baseline (speedup 1.0000x reference)
import jax
import jax.numpy as jnp
from jax.experimental import pallas as pl


def kernel(latents, embedding):
    raise NotImplementedError("write your pallas kernel here")


# speedup vs baseline: 1.1734x; 1.1734x over previous
import functools

import jax
import jax.numpy as jnp
from jax import lax
from jax.experimental import pallas as pl
from jax.experimental.pallas import tpu as pltpu
from jax.experimental.pallas import tpu_sc as plsc

_K = 1024
_D = 64
_N = 32768
_BETA = 0.25
_BN = 512


def _tc_body(lat_ref, emb_ref, inds_ref, loss_ref):
    lat = lat_ref[...]
    emb = emb_ref[...]
    e_sq = jnp.sum(emb * emb, axis=1)
    l_sq = jnp.sum(lat * lat, axis=1, keepdims=True)
    cross = lax.dot_general(
        lat, emb, (((1,), (1,)), ((), ())),
        preferred_element_type=jnp.float32,
        precision=lax.Precision.DEFAULT,
    )
    dist = l_sq + e_sq[None, :] - 2.0 * cross
    m = jnp.min(dist, axis=1, keepdims=True)
    kidx = lax.broadcasted_iota(jnp.int32, dist.shape, 1)
    idx = jnp.min(jnp.where(dist <= m, kidx, _K), axis=1)
    inds_ref[...] = idx
    loss_ref[...] = (1.0 + _BETA) * m[:, 0]


def _tc_distance_argmin(latents, embedding):
    grid = _N // _BN
    return pl.pallas_call(
        _tc_body,
        grid=(grid,),
        in_specs=[
            pl.BlockSpec((_BN, _D), lambda i: (i, 0)),
            pl.BlockSpec((_K, _D), lambda i: (0, 0)),
        ],
        out_specs=[
            pl.BlockSpec((_BN,), lambda i: (i,)),
            pl.BlockSpec((_BN,), lambda i: (i,)),
        ],
        out_shape=[
            jax.ShapeDtypeStruct((_N,), jnp.int32),
            jax.ShapeDtypeStruct((_N,), jnp.float32),
        ],
    )(latents, embedding)


_SC_CORES = 2
_SC_SUBCORES = 16
_NW = _SC_CORES * _SC_SUBCORES
_BPW = _N // _NW


@functools.lru_cache(maxsize=None)
def _make_sc_gather():
    @functools.partial(
        pl.kernel,
        mesh=plsc.VectorSubcoreMesh(core_axis_name="c", subcore_axis_name="s"),
        out_type=jax.ShapeDtypeStruct((_N, _D), jnp.float32),
        scratch_types=[
            pltpu.VMEM((_BPW,), jnp.int32),
            pltpu.VMEM((_BPW, _D), jnp.float32),
            pltpu.SemaphoreType.DMA,
        ],
        compiler_params=pltpu.CompilerParams(use_tc_tiling_on_sc=False),
    )
    def _sc_gather(idx_hbm, table_hbm, out_hbm, idx_v, rows_v, sem):
        wid = lax.axis_index("s") * _SC_CORES + lax.axis_index("c")
        base = wid * _BPW
        pltpu.sync_copy(idx_hbm.at[pl.ds(base, _BPW)], idx_v)
        pltpu.async_copy(table_hbm.at[idx_v], rows_v, sem).wait()
        pltpu.sync_copy(rows_v, out_hbm.at[pl.ds(base, _BPW)])

    return _sc_gather


def kernel(latents, embedding):
    inds, vq_loss = _tc_distance_argmin(latents, embedding)
    quantized = _make_sc_gather()(inds, embedding)
    return quantized, vq_loss


# speedup vs baseline: 1.7681x; 1.5068x over previous
import functools

import jax
import jax.numpy as jnp
from jax import lax
from jax.experimental import pallas as pl
from jax.experimental.pallas import tpu as pltpu
from jax.experimental.pallas import tpu_sc as plsc

_K = 1024
_D = 64
_N = 32768
_BETA = 0.25
_BN = 512


def _tc_body(lat_ref, emb_ref, inds_ref, loss_ref, e_sq_ref):
    @pl.when(pl.program_id(0) == 0)
    def _():
        emb0 = emb_ref[...]
        e_sq_ref[...] = jnp.sum(emb0 * emb0, axis=1)[None, :]

    lat = lat_ref[...]
    emb = emb_ref[...]
    l_sq = jnp.sum(lat * lat, axis=1, keepdims=True)
    cross = lax.dot_general(
        lat, emb, (((1,), (1,)), ((), ())),
        preferred_element_type=jnp.float32,
        precision=lax.Precision.DEFAULT,
    )
    dist = (l_sq + e_sq_ref[...]) - 2.0 * cross
    _C = 128
    m = dist[:, 0:_C]
    am = jnp.zeros((_BN, _C), jnp.int32)
    for c in range(1, _K // _C):
        d_c = dist[:, c * _C:(c + 1) * _C]
        lt = d_c < m
        m = jnp.where(lt, d_c, m)
        am = jnp.where(lt, c, am)
    mT = m.T
    amT = am.T
    mmin = jnp.min(mT, axis=0)
    lane = lax.broadcasted_iota(jnp.int32, (_C, _BN), 0)
    kfull = amT * _C + lane
    idx = jnp.min(jnp.where(mT <= mmin[None, :], kfull, _K), axis=0)
    inds_ref[...] = idx
    loss_ref[...] = (1.0 + _BETA) * mmin


def _tc_distance_argmin(latents, embedding):
    grid = _N // _BN
    return pl.pallas_call(
        _tc_body,
        grid=(grid,),
        in_specs=[
            pl.BlockSpec((_BN, _D), lambda i: (i, 0)),
            pl.BlockSpec((_K, _D), lambda i: (0, 0)),
        ],
        out_specs=[
            pl.BlockSpec((_BN,), lambda i: (i,)),
            pl.BlockSpec((_BN,), lambda i: (i,)),
        ],
        out_shape=[
            jax.ShapeDtypeStruct((_N,), jnp.int32),
            jax.ShapeDtypeStruct((_N,), jnp.float32),
        ],
        scratch_shapes=[pltpu.VMEM((1, _K), jnp.float32)],
    )(latents, embedding)


_SC_CORES = 2
_SC_SUBCORES = 16
_NW = _SC_CORES * _SC_SUBCORES
_BPW = _N // _NW


@functools.lru_cache(maxsize=None)
def _make_sc_gather():
    @functools.partial(
        pl.kernel,
        mesh=plsc.VectorSubcoreMesh(core_axis_name="c", subcore_axis_name="s"),
        out_type=jax.ShapeDtypeStruct((_N, _D), jnp.float32),
        scratch_types=[
            pltpu.VMEM((_BPW,), jnp.int32),
            pltpu.VMEM((_BPW, _D), jnp.float32),
            pltpu.SemaphoreType.DMA,
        ],
        compiler_params=pltpu.CompilerParams(use_tc_tiling_on_sc=False),
    )
    def _sc_gather(idx_hbm, table_hbm, out_hbm, idx_v, rows_v, sem):
        wid = lax.axis_index("s") * _SC_CORES + lax.axis_index("c")
        base = wid * _BPW
        pltpu.sync_copy(idx_hbm.at[pl.ds(base, _BPW)], idx_v)
        pltpu.async_copy(table_hbm.at[idx_v], rows_v, sem).wait()
        pltpu.sync_copy(rows_v, out_hbm.at[pl.ds(base, _BPW)])

    return _sc_gather


def kernel(latents, embedding):
    inds, vq_loss = _tc_distance_argmin(latents, embedding)
    quantized = _make_sc_gather()(inds, embedding)
    return quantized, vq_loss


# speedup vs baseline: 1.9773x; 1.1183x over previous
import functools

import jax
import jax.numpy as jnp
from jax import lax
from jax.experimental import pallas as pl
from jax.experimental.pallas import tpu as pltpu
from jax.experimental.pallas import tpu_sc as plsc

_K = 1024
_D = 64
_N = 32768
_BETA = 0.25
_BN = 1024


def _tc_body(lat_ref, emb_ref, inds_ref, loss_ref, e_sq_ref):
    @pl.when(pl.program_id(0) == 0)
    def _():
        emb0 = emb_ref[...]
        e_sq_ref[...] = jnp.sum(emb0 * emb0, axis=1)[None, :]

    lat = lat_ref[...]
    emb = emb_ref[...]
    l_sq = jnp.sum(lat * lat, axis=1, keepdims=True)
    cross = lax.dot_general(
        lat, emb, (((1,), (1,)), ((), ())),
        preferred_element_type=jnp.float32,
        precision=lax.Precision.DEFAULT,
    )
    dist = (l_sq + e_sq_ref[...]) - 2.0 * cross
    _C = 128
    m = dist[:, 0:_C]
    am = jnp.zeros((_BN, _C), jnp.int32)
    for c in range(1, _K // _C):
        d_c = dist[:, c * _C:(c + 1) * _C]
        lt = d_c < m
        m = jnp.where(lt, d_c, m)
        am = jnp.where(lt, c, am)
    mT = m.T
    amT = am.T
    mmin = jnp.min(mT, axis=0)
    lane = lax.broadcasted_iota(jnp.int32, (_C, _BN), 0)
    kfull = amT * _C + lane
    idx = jnp.min(jnp.where(mT <= mmin[None, :], kfull, _K), axis=0)
    inds_ref[...] = idx
    loss_ref[...] = (1.0 + _BETA) * mmin


def _tc_distance_argmin(latents, embedding):
    grid = _N // _BN
    return pl.pallas_call(
        _tc_body,
        grid=(grid,),
        in_specs=[
            pl.BlockSpec((_BN, _D), lambda i: (i, 0)),
            pl.BlockSpec((_K, _D), lambda i: (0, 0)),
        ],
        out_specs=[
            pl.BlockSpec((_BN,), lambda i: (i,)),
            pl.BlockSpec((_BN,), lambda i: (i,)),
        ],
        out_shape=[
            jax.ShapeDtypeStruct((_N,), jnp.int32),
            jax.ShapeDtypeStruct((_N,), jnp.float32),
        ],
        scratch_shapes=[pltpu.VMEM((1, _K), jnp.float32)],
    )(latents, embedding)


_SC_CORES = 2
_SC_SUBCORES = 16
_NW = _SC_CORES * _SC_SUBCORES
_BPW = _N // _NW


@functools.lru_cache(maxsize=None)
def _make_sc_gather():
    @functools.partial(
        pl.kernel,
        mesh=plsc.VectorSubcoreMesh(core_axis_name="c", subcore_axis_name="s"),
        out_type=jax.ShapeDtypeStruct((_N, _D), jnp.float32),
        scratch_types=[
            pltpu.VMEM((_BPW,), jnp.int32),
            pltpu.VMEM((_BPW, _D), jnp.float32),
            pltpu.SemaphoreType.DMA,
        ],
        compiler_params=pltpu.CompilerParams(use_tc_tiling_on_sc=False),
    )
    def _sc_gather(idx_hbm, table_hbm, out_hbm, idx_v, rows_v, sem):
        wid = lax.axis_index("s") * _SC_CORES + lax.axis_index("c")
        base = wid * _BPW
        pltpu.sync_copy(idx_hbm.at[pl.ds(base, _BPW)], idx_v)
        pltpu.async_copy(table_hbm.at[idx_v], rows_v, sem).wait()
        pltpu.sync_copy(rows_v, out_hbm.at[pl.ds(base, _BPW)])

    return _sc_gather


def kernel(latents, embedding):
    inds, vq_loss = _tc_distance_argmin(latents, embedding)
    quantized = _make_sc_gather()(inds, embedding)
    return quantized, vq_loss
